# baseline (speedup 1.0000x reference)
import functools
import math

import jax
import jax.numpy as jnp
from jax import lax
from jax.experimental import pallas as pl
from jax.experimental.pallas import tpu as pltpu
from jax.experimental.pallas import tpu_sc as plsc

N = 10000
E = 100000
B = 8192
D = 768
H = 4
C = 256
HC = H * C
FH = 256
HD = 6

NPAD = 10240
ROWB = 256
BB = 512
XW = 128
SQRT2 = math.sqrt(2.0)


def _gelu(x):
    return 0.5 * x * (1.0 + lax.erf(x / SQRT2))


def _k1_body(gx_ref, wg_ref, att_ref, xt_ref, acat_ref):
    xb = jnp.dot(gx_ref[...].astype(jnp.bfloat16),
                 wg_ref[...].astype(jnp.bfloat16),
                 preferred_element_type=jnp.float32)
    for g in range(8):
        xt_ref[g] = xb[:, g * 128:(g + 1) * 128]
    for h in range(H):
        xs = xb[:, h * C:(h + 1) * C]
        asrc = jnp.sum(xs * att_ref[0, h][None, :], axis=1)
        adst = jnp.sum(xs * att_ref[1, h][None, :], axis=1)
        acat_ref[h] = jnp.stack([asrc, adst], axis=1)


def _k1(gx_pad, w_gat, att):
    nblocks = NPAD // ROWB
    return pl.pallas_call(
        _k1_body,
        grid=(nblocks,),
        in_specs=[
            pl.BlockSpec((ROWB, D), lambda i: (i, 0)),
            pl.BlockSpec((D, HC), lambda i: (0, 0)),
            pl.BlockSpec((2, H, C), lambda i: (0, 0, 0)),
        ],
        out_specs=[
            pl.BlockSpec((8, ROWB, XW), lambda i: (0, i, 0)),
            pl.BlockSpec((4, ROWB, 2), lambda i: (0, i, 0)),
        ],
        out_shape=[
            jax.ShapeDtypeStruct((8, NPAD, XW), jnp.float32),
            jax.ShapeDtypeStruct((4, NPAD, 2), jnp.float32),
        ],
    )(gx_pad, w_gat, att)


def _k3_body(sem_ref, outg_ref, den_ref, heur_ref,
             wsem1_ref, bsem1_ref, wsem2_ref, bsem2_ref,
             bgat_ref, wstr_ref, bstr_ref,
             wheur_ref, bheur_ref, wf1_ref, bf1_ref, wf2_ref, bf2_ref,
             logit_ref, out_ref):
    h = _gelu(jnp.dot(sem_ref[...], wsem1_ref[...],
                      preferred_element_type=jnp.float32) + bsem1_ref[...])
    h_sem = jnp.dot(h, wsem2_ref[...],
                    preferred_element_type=jnp.float32) + bsem2_ref[...]

    den = den_ref[0] + den_ref[1]
    hstr_acc = jnp.zeros((BB, FH), dtype=jnp.float32)
    for g in range(8):
        rows = outg_ref[g]
        d = den[:, (g // 2):(g // 2) + 1]
        hg = rows / (d + 1e-16) + bgat_ref[g][None, :]
        hg = jnp.where(hg > 0, hg, jnp.exp(hg) - 1.0)
        hstr_acc = hstr_acc + jnp.dot(hg, wstr_ref[g],
                                      preferred_element_type=jnp.float32)
    h_str = _gelu(hstr_acc + bstr_ref[...])

    fu = (jnp.dot(h_sem, wf1_ref[0], preferred_element_type=jnp.float32)
          + jnp.dot(h_str, wf1_ref[1], preferred_element_type=jnp.float32)
          + bf1_ref[...])
    fu = _gelu(fu)
    fused = jnp.sum(fu * wf2_ref[...], axis=1, keepdims=True) + bf2_ref[0, 0]

    heur_score = jnp.sum(heur_ref[...] * wheur_ref[...], axis=1,
                         keepdims=True) + bheur_ref[0, 0]
    wgt = 1.0 / (1.0 + jnp.exp(-logit_ref[0, 0]))
    out_ref[...] = wgt * heur_score + (1.0 - wgt) * fused


def _k3(sem, out_gath, den_gath, heur_pad, wsem1, bsem1, wsem2, bsem2,
        bgat, wstr, bstr, wheur, bheur, wf1, bf1, wf2, bf2, logit):
    nblocks = B // BB
    res = lambda *s: pl.BlockSpec(s, lambda i: (0,) * len(s))
    return pl.pallas_call(
        _k3_body,
        grid=(nblocks,),
        in_specs=[
            pl.BlockSpec((BB, 2 * D), lambda i: (i, 0)),
            pl.BlockSpec((8, BB, XW), lambda i: (0, i, 0)),
            pl.BlockSpec((2, BB, 128), lambda i: (0, i, 0)),
            pl.BlockSpec((BB, 8), lambda i: (i, 0)),
            res(2 * D, FH), res(1, FH), res(FH, FH), res(1, FH),
            res(8, 128), res(8, 128, FH), res(1, FH),
            res(1, 8), res(1, 1), res(2, FH, FH), res(1, FH), res(1, FH),
            res(1, 1), res(1, 1),
        ],
        out_specs=pl.BlockSpec((BB, 1), lambda i: (i, 0)),
        out_shape=jax.ShapeDtypeStruct((B, 1), jnp.float32),
    )(sem, out_gath, den_gath, heur_pad, wsem1, bsem1, wsem2, bsem2,
      bgat, wstr, bstr, wheur, bheur, wf1, bf1, wf2, bf2, logit)


EP = 102400
EPA = EP // 32
EPB = EP // 16
CHK = 128
NPT = NPAD // 16


def _sc_mesh():
    return plsc.VectorSubcoreMesh(core_axis_name="c", subcore_axis_name="s",
                                  num_cores=2, num_subcores=16)


EPH = EP // 8


def _phase_a_body(acat_hbm, src_hbm, dst_hbm, ew_hbm, k_hbm, nidx_hbm,
                  ex_hbm, den_hbm,
                  acat_v, srcb0, ewb0, dstb0, srcb1, ewb1, dstb1,
                  kv, exb, dsrc, idxb, zb,
                  acc, sem, isem0, isem1):
    c = lax.axis_index("c")
    s = lax.axis_index("s")
    hh = s % 4
    slot = c * 4 + (s // 4)
    base = slot * EPH
    iota = lax.iota(jnp.int32, 16)

    pltpu.sync_copy(acat_hbm.at[pl.ds(hh * (NPAD * 2), NPAD * 2)], acat_v)
    pltpu.sync_copy(k_hbm, kv)
    kva = kv[pl.ds(0, 16)]
    kh = kva[0]
    for h in range(1, 4):
        kh = jnp.where(hh == h, kva[h], kh)

    zero16 = jnp.zeros((16,), jnp.float32)

    def _zrow(i, carry):
        for jz in range(8):
            dsrc[i, pl.ds(jz * 16, 16)] = zero16
        return carry
    lax.fori_loop(0, CHK, _zrow, 0)

    def _zzb(i, carry):
        for jz in range(8):
            zb[i, pl.ds(jz * 16, 16)] = zero16
        return carry
    lax.fori_loop(0, 16, _zzb, 0)
    row0 = s * NPT
    for t in range(NPT // 16):
        pltpu.sync_copy(zb, acc.at[pl.ds(row0 + t * 16, 16)])
    plsc.subcore_barrier()

    def _fetch(off, sb, eb, db, isem):
        pltpu.async_copy(src_hbm.at[pl.ds(off, CHK)], sb, isem)
        pltpu.async_copy(ew_hbm.at[pl.ds(off, CHK)], eb, isem)
        pltpu.async_copy(dst_hbm.at[pl.ds(off, CHK)], db, isem)

    def _wait3(sb, eb, db, isem):
        pltpu.make_async_copy(src_hbm.at[pl.ds(base, CHK)], sb, isem).wait()
        pltpu.make_async_copy(ew_hbm.at[pl.ds(base, CHK)], eb, isem).wait()
        pltpu.make_async_copy(dst_hbm.at[pl.ds(base, CHK)], db, isem).wait()

    def _compute(off, sb, eb, db):
        for j in range(8):
            src16 = sb[pl.ds(j * 16, 16)] * 2
            w16 = eb[pl.ds(j * 16, 16)]
            dst16 = db[pl.ds(j * 16, 16)] * 2 + 1
            asrc = plsc.load_gather(acat_v, [src16])
            adst = plsc.load_gather(acat_v, [dst16])
            al = asrc + adst + w16 * kh
            al = jnp.where(al >= 0, al, 0.2 * al)
            exh = jnp.exp(al)
            exb[pl.ds(j * 16, 16)] = exh
            for l in range(16):
                v = jnp.where(iota == hh, exh[l], zero16)
                dsrc[j * 16 + l, pl.ds(0, 16)] = v
        pltpu.sync_copy(exb, ex_hbm.at[hh, pl.ds(off, CHK)])
        pltpu.sync_copy(dsrc, acc.at[db], add=True)

    _fetch(base, srcb0, ewb0, dstb0, isem0)
    _fetch(base + CHK, srcb1, ewb1, dstb1, isem1)
    npair = EPH // CHK // 2

    def _pair(i, carry):
        off0 = base + (2 * i) * CHK
        _wait3(srcb0, ewb0, dstb0, isem0)
        _compute(off0, srcb0, ewb0, dstb0)

        @pl.when(i + 1 < npair)
        def _pf0():
            _fetch(off0 + 2 * CHK, srcb0, ewb0, dstb0, isem0)
        _wait3(srcb1, ewb1, dstb1, isem1)
        _compute(off0 + CHK, srcb1, ewb1, dstb1)

        @pl.when(i + 1 < npair)
        def _pf1():
            _fetch(off0 + 3 * CHK, srcb1, ewb1, dstb1, isem1)
        return carry
    lax.fori_loop(0, npair, _pair, 0)
    plsc.subcore_barrier()

    for q in range(4):
        boff = s * 512 + q * CHK
        pltpu.sync_copy(nidx_hbm.at[pl.ds(boff, CHK)], idxb)
        pltpu.async_copy(acc.at[idxb], dsrc, sem).wait()
        pltpu.sync_copy(dsrc, den_hbm.at[c, pl.ds(boff, CHK)])


@functools.cache
def _phase_a():
    return pl.kernel(
        _phase_a_body,
        out_type=[jax.ShapeDtypeStruct((4, EP), jnp.float32),
                  jax.ShapeDtypeStruct((2, B, 128), jnp.float32)],
        mesh=_sc_mesh(),
        compiler_params=pltpu.CompilerParams(needs_layout_passes=False),
        scratch_types=[
            pltpu.VMEM((NPAD * 2,), jnp.float32),
            pltpu.VMEM((CHK,), jnp.int32),
            pltpu.VMEM((CHK,), jnp.float32),
            pltpu.VMEM((CHK,), jnp.int32),
            pltpu.VMEM((CHK,), jnp.int32),
            pltpu.VMEM((CHK,), jnp.float32),
            pltpu.VMEM((CHK,), jnp.int32),
            pltpu.VMEM((16,), jnp.float32),
            pltpu.VMEM((CHK,), jnp.float32),
            pltpu.VMEM((CHK, 128), jnp.float32),
            pltpu.VMEM((CHK,), jnp.int32),
            pltpu.VMEM((16, 128), jnp.float32),
            pltpu.VMEM_SHARED((NPAD, 128), jnp.float32),
            pltpu.SemaphoreType.DMA,
            pltpu.SemaphoreType.DMA,
            pltpu.SemaphoreType.DMA,
        ],
    )


def _phase_b_body(xt_hbm, src_hbm, dst_hbm, ex_hbm, nidx_hbm,
                  outg_hbm,
                  adjstage, dstb0, dstb1, exb0, exb1, rows0, rows1, idxb, zb,
                  acc, gsem0, gsem1, ssem0, ssem1):
    c = lax.axis_index("c")
    s = lax.axis_index("s")
    base = s * EPB
    row0 = s * NPT

    def _scale(rows, exb):
        def _sc(jg, cy):
            ex16 = exb[pl.ds(jg * 16, 16)]
            kbase = jg * 16
            for l in range(16):
                ev = ex16[l]
                for j2 in range(8):
                    sl = pl.ds(j2 * 16, 16)
                    rows[kbase + l, sl] = rows[kbase + l, sl] * ev
            return cy
        lax.fori_loop(0, CHK // 16, _sc, 0)

    for fb in range(4):
        g = c * 4 + fb
        h = c * 2 + (fb // 2)
        goff = g * NPAD
        plsc.subcore_barrier()
        if fb == 0:
            def _zzb(i, carry):
                for jz in range(8):
                    zb[i, pl.ds(jz * 16, 16)] = jnp.zeros((16,), jnp.float32)
                return carry
            lax.fori_loop(0, 64, _zzb, 0)
        for t in range(NPT // 64):
            pltpu.sync_copy(zb, acc.at[pl.ds(row0 + t * 64, 64)])
        pltpu.sync_copy(src_hbm.at[pl.ds(base, EPB)], adjstage)

        def _adj(i, cy):
            sl = pl.ds(i * 16, 16)
            adjstage[sl] = adjstage[sl] + goff
            return cy
        lax.fori_loop(0, EPB // 16, _adj, 0)
        plsc.subcore_barrier()

        def _pair(i, carry):
            @pl.when(i > 0)
            def _drain():
                pltpu.make_async_copy(rows0, acc.at[dstb0], ssem0).wait()
                pltpu.make_async_copy(rows1, acc.at[dstb1], ssem1).wait()
            off0 = (2 * i) * CHK
            off1 = off0 + CHK
            pltpu.sync_copy(dst_hbm.at[pl.ds(base + off0, CHK)], dstb0)
            pltpu.sync_copy(ex_hbm.at[h, pl.ds(base + off0, CHK)], exb0)
            d0 = pltpu.async_copy(
                xt_hbm.at[adjstage.at[pl.ds(off0, CHK)]], rows0, gsem0)
            pltpu.sync_copy(dst_hbm.at[pl.ds(base + off1, CHK)], dstb1)
            pltpu.sync_copy(ex_hbm.at[h, pl.ds(base + off1, CHK)], exb1)
            d1 = pltpu.async_copy(
                xt_hbm.at[adjstage.at[pl.ds(off1, CHK)]], rows1, gsem1)
            d0.wait()
            _scale(rows0, exb0)
            pltpu.async_copy(rows0, acc.at[dstb0], ssem0, add=True)
            d1.wait()
            _scale(rows1, exb1)
            pltpu.async_copy(rows1, acc.at[dstb1], ssem1, add=True)
            return carry
        lax.fori_loop(0, EPB // CHK // 2, _pair, 0)
        pltpu.make_async_copy(rows0, acc.at[dstb0], ssem0).wait()
        pltpu.make_async_copy(rows1, acc.at[dstb1], ssem1).wait()
        plsc.subcore_barrier()

        for q in range(4):
            boff = s * 512 + q * CHK
            pltpu.sync_copy(nidx_hbm.at[pl.ds(boff, CHK)], idxb)
            pltpu.async_copy(acc.at[idxb], rows0, gsem0).wait()
            pltpu.sync_copy(rows0, outg_hbm.at[g, pl.ds(boff, CHK)])


@functools.cache
def _phase_b():
    return pl.kernel(
        _phase_b_body,
        out_type=jax.ShapeDtypeStruct((8, B, XW), jnp.float32),
        mesh=_sc_mesh(),
        compiler_params=pltpu.CompilerParams(needs_layout_passes=False),
        scratch_types=[
            pltpu.VMEM((EPB,), jnp.int32),
            pltpu.VMEM((CHK,), jnp.int32),
            pltpu.VMEM((CHK,), jnp.int32),
            pltpu.VMEM((CHK,), jnp.float32),
            pltpu.VMEM((CHK,), jnp.float32),
            pltpu.VMEM((CHK, XW), jnp.float32),
            pltpu.VMEM((CHK, XW), jnp.float32),
            pltpu.VMEM((CHK,), jnp.int32),
            pltpu.VMEM((64, XW), jnp.float32),
            pltpu.VMEM_SHARED((NPAD, XW), jnp.float32),
            pltpu.SemaphoreType.DMA,
            pltpu.SemaphoreType.DMA,
            pltpu.SemaphoreType.DMA,
            pltpu.SemaphoreType.DMA,
        ],
    )


def kernel(semantic_embeddings, graph_x, graph_edge_index, graph_edge_weight,
           heuristic_features, node_indices, W_sem1, b_sem1, W_sem2, b_sem2,
           W_gat, att_src, att_dst, W_edge, att_edge, b_gat, W_str, b_str,
           W_heur, b_heur, W_f1, b_f1, W_f2, b_f2, heur_logit):
    gx_pad = jnp.pad(graph_x, ((0, NPAD - N), (0, 0)))
    att = jnp.stack([att_src[0], att_dst[0]])
    xt, acat = _k1(gx_pad, W_gat, att)

    kvec = jnp.sum(W_edge.reshape(H, C) * att_edge[0], axis=1)
    pad = EP - E
    srcp = jnp.pad(graph_edge_index[0], (0, pad))
    dstp = jnp.pad(graph_edge_index[1], (0, pad), constant_values=NPAD - 1)
    ewp = jnp.pad(graph_edge_weight[:, 0], (0, pad))
    kpad = jnp.pad(kvec, (0, 12))
    ex, den_gath = _phase_a()(acat.reshape(NPAD * 8), srcp, dstp, ewp, kpad,
                              node_indices)
    out_gath = _phase_b()(xt.reshape(8 * NPAD, XW), srcp, dstp, ex,
                          node_indices)

    heur_pad = jnp.pad(heuristic_features, ((0, 0), (0, 8 - HD)))
    wheur_pad = jnp.pad(W_heur[:, 0], (0, 8 - HD)).reshape(1, 8)
    wstr = W_str.reshape(8, 128, FH)
    wf1 = W_f1.reshape(2, FH, FH)
    scores = _k3(
        semantic_embeddings, out_gath, den_gath, heur_pad,
        W_sem1, b_sem1.reshape(1, FH), W_sem2, b_sem2.reshape(1, FH),
        b_gat.reshape(8, 128), wstr, b_str.reshape(1, FH),
        wheur_pad, b_heur.reshape(1, 1), wf1, b_f1.reshape(1, FH),
        W_f2.reshape(1, FH), b_f2.reshape(1, 1),
        heur_logit.reshape(1, 1))
    return scores


# speedup vs baseline: 1.0009x; 1.0009x over previous
import functools
import math

import jax
import jax.numpy as jnp
from jax import lax
from jax.experimental import pallas as pl
from jax.experimental.pallas import tpu as pltpu
from jax.experimental.pallas import tpu_sc as plsc

N = 10000
E = 100000
B = 8192
D = 768
H = 4
C = 256
HC = H * C
FH = 256
HD = 6

NPAD = 10240
ROWB = 256
BB = 512
XW = 128
SQRT2 = math.sqrt(2.0)


def _gelu(x):
    return 0.5 * x * (1.0 + lax.erf(x / SQRT2))


def _k1_body(gx_ref, wg_ref, att_ref, xt_ref, acat_ref):
    xb = jnp.dot(gx_ref[...], wg_ref[...], preferred_element_type=jnp.float32)
    for g in range(8):
        xt_ref[g] = xb[:, g * 128:(g + 1) * 128]
    for h in range(H):
        xs = xb[:, h * C:(h + 1) * C]
        asrc = jnp.sum(xs * att_ref[0, h][None, :], axis=1)
        adst = jnp.sum(xs * att_ref[1, h][None, :], axis=1)
        acat_ref[h] = jnp.stack([asrc, adst], axis=1)


def _k1(gx_pad, w_gat, att):
    nblocks = NPAD // ROWB
    return pl.pallas_call(
        _k1_body,
        grid=(nblocks,),
        in_specs=[
            pl.BlockSpec((ROWB, D), lambda i: (i, 0)),
            pl.BlockSpec((D, HC), lambda i: (0, 0)),
            pl.BlockSpec((2, H, C), lambda i: (0, 0, 0)),
        ],
        out_specs=[
            pl.BlockSpec((8, ROWB, XW), lambda i: (0, i, 0)),
            pl.BlockSpec((4, ROWB, 2), lambda i: (0, i, 0)),
        ],
        out_shape=[
            jax.ShapeDtypeStruct((8, NPAD, XW), jnp.float32),
            jax.ShapeDtypeStruct((4, NPAD, 2), jnp.float32),
        ],
    )(gx_pad, w_gat, att)


def _k3_body(sem_ref, outg_ref, den_ref, heur_ref,
             wsem1_ref, bsem1_ref, wsem2_ref, bsem2_ref,
             bgat_ref, wstr_ref, bstr_ref,
             wheur_ref, bheur_ref, wf1_ref, bf1_ref, wf2_ref, bf2_ref,
             logit_ref, out_ref):
    h = _gelu(jnp.dot(sem_ref[...], wsem1_ref[...],
                      preferred_element_type=jnp.float32) + bsem1_ref[...])
    h_sem = jnp.dot(h, wsem2_ref[...],
                    preferred_element_type=jnp.float32) + bsem2_ref[...]

    den = den_ref[0] + den_ref[1]
    hstr_acc = jnp.zeros((BB, FH), dtype=jnp.float32)
    for g in range(8):
        rows = outg_ref[g]
        d = den[:, (g // 2):(g // 2) + 1]
        hg = rows / (d + 1e-16) + bgat_ref[g][None, :]
        hg = jnp.where(hg > 0, hg, jnp.exp(hg) - 1.0)
        hstr_acc = hstr_acc + jnp.dot(hg, wstr_ref[g],
                                      preferred_element_type=jnp.float32)
    h_str = _gelu(hstr_acc + bstr_ref[...])

    fu = (jnp.dot(h_sem, wf1_ref[0], preferred_element_type=jnp.float32)
          + jnp.dot(h_str, wf1_ref[1], preferred_element_type=jnp.float32)
          + bf1_ref[...])
    fu = _gelu(fu)
    fused = jnp.sum(fu * wf2_ref[...], axis=1, keepdims=True) + bf2_ref[0, 0]

    heur_score = jnp.sum(heur_ref[...] * wheur_ref[...], axis=1,
                         keepdims=True) + bheur_ref[0, 0]
    wgt = 1.0 / (1.0 + jnp.exp(-logit_ref[0, 0]))
    out_ref[...] = wgt * heur_score + (1.0 - wgt) * fused


def _k3(sem, out_gath, den_gath, heur_pad, wsem1, bsem1, wsem2, bsem2,
        bgat, wstr, bstr, wheur, bheur, wf1, bf1, wf2, bf2, logit):
    nblocks = B // BB
    res = lambda *s: pl.BlockSpec(s, lambda i: (0,) * len(s))
    return pl.pallas_call(
        _k3_body,
        grid=(nblocks,),
        in_specs=[
            pl.BlockSpec((BB, 2 * D), lambda i: (i, 0)),
            pl.BlockSpec((8, BB, XW), lambda i: (0, i, 0)),
            pl.BlockSpec((2, BB, 128), lambda i: (0, i, 0)),
            pl.BlockSpec((BB, 8), lambda i: (i, 0)),
            res(2 * D, FH), res(1, FH), res(FH, FH), res(1, FH),
            res(8, 128), res(8, 128, FH), res(1, FH),
            res(1, 8), res(1, 1), res(2, FH, FH), res(1, FH), res(1, FH),
            res(1, 1), res(1, 1),
        ],
        out_specs=pl.BlockSpec((BB, 1), lambda i: (i, 0)),
        out_shape=jax.ShapeDtypeStruct((B, 1), jnp.float32),
    )(sem, out_gath, den_gath, heur_pad, wsem1, bsem1, wsem2, bsem2,
      bgat, wstr, bstr, wheur, bheur, wf1, bf1, wf2, bf2, logit)


EP = 102400
EPA = EP // 32
EPB = EP // 16
CHK = 128
NPT = NPAD // 16


def _sc_mesh():
    return plsc.VectorSubcoreMesh(core_axis_name="c", subcore_axis_name="s",
                                  num_cores=2, num_subcores=16)


EPH = EP // 8


def _phase_a_body(acat_hbm, src_hbm, dst_hbm, ew_hbm, k_hbm, nidx_hbm,
                  ex_hbm, den_hbm,
                  acat_v, srcb0, ewb0, dstb0, srcb1, ewb1, dstb1,
                  kv, exb, dsrc, idxb, zb,
                  acc, sem, isem0, isem1):
    c = lax.axis_index("c")
    s = lax.axis_index("s")
    hh = s % 4
    slot = c * 4 + (s // 4)
    base = slot * EPH
    iota = lax.iota(jnp.int32, 16)

    pltpu.sync_copy(acat_hbm.at[pl.ds(hh * (NPAD * 2), NPAD * 2)], acat_v)
    pltpu.sync_copy(k_hbm, kv)
    kva = kv[pl.ds(0, 16)]
    kh = kva[0]
    for h in range(1, 4):
        kh = jnp.where(hh == h, kva[h], kh)

    zero16 = jnp.zeros((16,), jnp.float32)

    def _zrow(i, carry):
        for jz in range(8):
            dsrc[i, pl.ds(jz * 16, 16)] = zero16
        return carry
    lax.fori_loop(0, CHK, _zrow, 0)

    def _zzb(i, carry):
        for jz in range(8):
            zb[i, pl.ds(jz * 16, 16)] = zero16
        return carry
    lax.fori_loop(0, 16, _zzb, 0)
    row0 = s * NPT
    for t in range(NPT // 16):
        pltpu.sync_copy(zb, acc.at[pl.ds(row0 + t * 16, 16)])
    plsc.subcore_barrier()

    def _fetch(off, sb, eb, db, isem):
        pltpu.async_copy(src_hbm.at[pl.ds(off, CHK)], sb, isem)
        pltpu.async_copy(ew_hbm.at[pl.ds(off, CHK)], eb, isem)
        pltpu.async_copy(dst_hbm.at[pl.ds(off, CHK)], db, isem)

    def _wait3(sb, eb, db, isem):
        pltpu.make_async_copy(src_hbm.at[pl.ds(base, CHK)], sb, isem).wait()
        pltpu.make_async_copy(ew_hbm.at[pl.ds(base, CHK)], eb, isem).wait()
        pltpu.make_async_copy(dst_hbm.at[pl.ds(base, CHK)], db, isem).wait()

    def _compute(off, sb, eb, db):
        for j in range(8):
            src16 = sb[pl.ds(j * 16, 16)] * 2
            w16 = eb[pl.ds(j * 16, 16)]
            dst16 = db[pl.ds(j * 16, 16)] * 2 + 1
            asrc = plsc.load_gather(acat_v, [src16])
            adst = plsc.load_gather(acat_v, [dst16])
            al = asrc + adst + w16 * kh
            al = jnp.where(al >= 0, al, 0.2 * al)
            exh = jnp.exp(al)
            exb[pl.ds(j * 16, 16)] = exh
            for l in range(16):
                v = jnp.where(iota == hh, exh[l], zero16)
                dsrc[j * 16 + l, pl.ds(0, 16)] = v
        pltpu.sync_copy(exb, ex_hbm.at[hh, pl.ds(off, CHK)])
        pltpu.sync_copy(dsrc, acc.at[db], add=True)

    _fetch(base, srcb0, ewb0, dstb0, isem0)
    _fetch(base + CHK, srcb1, ewb1, dstb1, isem1)
    npair = EPH // CHK // 2

    def _pair(i, carry):
        off0 = base + (2 * i) * CHK
        _wait3(srcb0, ewb0, dstb0, isem0)
        _compute(off0, srcb0, ewb0, dstb0)

        @pl.when(i + 1 < npair)
        def _pf0():
            _fetch(off0 + 2 * CHK, srcb0, ewb0, dstb0, isem0)
        _wait3(srcb1, ewb1, dstb1, isem1)
        _compute(off0 + CHK, srcb1, ewb1, dstb1)

        @pl.when(i + 1 < npair)
        def _pf1():
            _fetch(off0 + 3 * CHK, srcb1, ewb1, dstb1, isem1)
        return carry
    lax.fori_loop(0, npair, _pair, 0)
    plsc.subcore_barrier()

    for q in range(4):
        boff = s * 512 + q * CHK
        pltpu.sync_copy(nidx_hbm.at[pl.ds(boff, CHK)], idxb)
        pltpu.async_copy(acc.at[idxb], dsrc, sem).wait()
        pltpu.sync_copy(dsrc, den_hbm.at[c, pl.ds(boff, CHK)])


@functools.cache
def _phase_a():
    return pl.kernel(
        _phase_a_body,
        out_type=[jax.ShapeDtypeStruct((4, EP), jnp.float32),
                  jax.ShapeDtypeStruct((2, B, 128), jnp.float32)],
        mesh=_sc_mesh(),
        compiler_params=pltpu.CompilerParams(needs_layout_passes=False),
        scratch_types=[
            pltpu.VMEM((NPAD * 2,), jnp.float32),
            pltpu.VMEM((CHK,), jnp.int32),
            pltpu.VMEM((CHK,), jnp.float32),
            pltpu.VMEM((CHK,), jnp.int32),
            pltpu.VMEM((CHK,), jnp.int32),
            pltpu.VMEM((CHK,), jnp.float32),
            pltpu.VMEM((CHK,), jnp.int32),
            pltpu.VMEM((16,), jnp.float32),
            pltpu.VMEM((CHK,), jnp.float32),
            pltpu.VMEM((CHK, 128), jnp.float32),
            pltpu.VMEM((CHK,), jnp.int32),
            pltpu.VMEM((16, 128), jnp.float32),
            pltpu.VMEM_SHARED((NPAD, 128), jnp.float32),
            pltpu.SemaphoreType.DMA,
            pltpu.SemaphoreType.DMA,
            pltpu.SemaphoreType.DMA,
        ],
    )


def _phase_b_body(xt_hbm, src_hbm, dst_hbm, ex_hbm, nidx_hbm,
                  outg_hbm,
                  adjstage, dstb0, dstb1, exb0, exb1, rows0, rows1, idxb, zb,
                  acc, gsem0, gsem1, ssem0, ssem1):
    c = lax.axis_index("c")
    s = lax.axis_index("s")
    base = s * EPB
    row0 = s * NPT

    def _scale(rows, exb):
        def _sc(jg, cy):
            ex16 = exb[pl.ds(jg * 16, 16)]
            kbase = jg * 16
            for l in range(16):
                ev = ex16[l]
                for j2 in range(8):
                    sl = pl.ds(j2 * 16, 16)
                    rows[kbase + l, sl] = rows[kbase + l, sl] * ev
            return cy
        lax.fori_loop(0, CHK // 16, _sc, 0)

    for fb in range(4):
        g = c * 4 + fb
        h = c * 2 + (fb // 2)
        goff = g * NPAD
        plsc.subcore_barrier()
        if fb == 0:
            def _zzb(i, carry):
                for jz in range(8):
                    zb[i, pl.ds(jz * 16, 16)] = jnp.zeros((16,), jnp.float32)
                return carry
            lax.fori_loop(0, 64, _zzb, 0)
        for t in range(NPT // 64):
            pltpu.sync_copy(zb, acc.at[pl.ds(row0 + t * 64, 64)])
        pltpu.sync_copy(src_hbm.at[pl.ds(base, EPB)], adjstage)

        def _adj(i, cy):
            sl = pl.ds(i * 16, 16)
            adjstage[sl] = adjstage[sl] + goff
            return cy
        lax.fori_loop(0, EPB // 16, _adj, 0)
        plsc.subcore_barrier()

        def _pair(i, carry):
            @pl.when(i > 0)
            def _drain():
                pltpu.make_async_copy(rows0, acc.at[dstb0], ssem0).wait()
                pltpu.make_async_copy(rows1, acc.at[dstb1], ssem1).wait()
            off0 = (2 * i) * CHK
            off1 = off0 + CHK
            pltpu.sync_copy(dst_hbm.at[pl.ds(base + off0, CHK)], dstb0)
            pltpu.sync_copy(ex_hbm.at[h, pl.ds(base + off0, CHK)], exb0)
            d0 = pltpu.async_copy(
                xt_hbm.at[adjstage.at[pl.ds(off0, CHK)]], rows0, gsem0)
            pltpu.sync_copy(dst_hbm.at[pl.ds(base + off1, CHK)], dstb1)
            pltpu.sync_copy(ex_hbm.at[h, pl.ds(base + off1, CHK)], exb1)
            d1 = pltpu.async_copy(
                xt_hbm.at[adjstage.at[pl.ds(off1, CHK)]], rows1, gsem1)
            d0.wait()
            _scale(rows0, exb0)
            pltpu.async_copy(rows0, acc.at[dstb0], ssem0, add=True)
            d1.wait()
            _scale(rows1, exb1)
            pltpu.async_copy(rows1, acc.at[dstb1], ssem1, add=True)
            return carry
        lax.fori_loop(0, EPB // CHK // 2, _pair, 0)
        pltpu.make_async_copy(rows0, acc.at[dstb0], ssem0).wait()
        pltpu.make_async_copy(rows1, acc.at[dstb1], ssem1).wait()
        plsc.subcore_barrier()

        for q in range(4):
            boff = s * 512 + q * CHK
            pltpu.sync_copy(nidx_hbm.at[pl.ds(boff, CHK)], idxb)
            pltpu.async_copy(acc.at[idxb], rows0, gsem0).wait()
            pltpu.sync_copy(rows0, outg_hbm.at[g, pl.ds(boff, CHK)])


@functools.cache
def _phase_b():
    return pl.kernel(
        _phase_b_body,
        out_type=jax.ShapeDtypeStruct((8, B, XW), jnp.float32),
        mesh=_sc_mesh(),
        compiler_params=pltpu.CompilerParams(needs_layout_passes=False),
        scratch_types=[
            pltpu.VMEM((EPB,), jnp.int32),
            pltpu.VMEM((CHK,), jnp.int32),
            pltpu.VMEM((CHK,), jnp.int32),
            pltpu.VMEM((CHK,), jnp.float32),
            pltpu.VMEM((CHK,), jnp.float32),
            pltpu.VMEM((CHK, XW), jnp.float32),
            pltpu.VMEM((CHK, XW), jnp.float32),
            pltpu.VMEM((CHK,), jnp.int32),
            pltpu.VMEM((64, XW), jnp.float32),
            pltpu.VMEM_SHARED((NPAD, XW), jnp.float32),
            pltpu.SemaphoreType.DMA,
            pltpu.SemaphoreType.DMA,
            pltpu.SemaphoreType.DMA,
            pltpu.SemaphoreType.DMA,
        ],
    )


def kernel(semantic_embeddings, graph_x, graph_edge_index, graph_edge_weight,
           heuristic_features, node_indices, W_sem1, b_sem1, W_sem2, b_sem2,
           W_gat, att_src, att_dst, W_edge, att_edge, b_gat, W_str, b_str,
           W_heur, b_heur, W_f1, b_f1, W_f2, b_f2, heur_logit):
    gx_pad = jnp.pad(graph_x, ((0, NPAD - N), (0, 0)))
    att = jnp.stack([att_src[0], att_dst[0]])
    xt, acat = _k1(gx_pad, W_gat, att)

    kvec = jnp.sum(W_edge.reshape(H, C) * att_edge[0], axis=1)
    pad = EP - E
    srcp = jnp.pad(graph_edge_index[0], (0, pad))
    dstp = jnp.pad(graph_edge_index[1], (0, pad), constant_values=NPAD - 1)
    ewp = jnp.pad(graph_edge_weight[:, 0], (0, pad))
    kpad = jnp.pad(kvec, (0, 12))
    ex, den_gath = _phase_a()(acat.reshape(NPAD * 8), srcp, dstp, ewp, kpad,
                              node_indices)
    out_gath = _phase_b()(xt.reshape(8 * NPAD, XW), srcp, dstp, ex,
                          node_indices)

    heur_pad = jnp.pad(heuristic_features, ((0, 0), (0, 8 - HD)))
    wheur_pad = jnp.pad(W_heur[:, 0], (0, 8 - HD)).reshape(1, 8)
    wstr = W_str.reshape(8, 128, FH)
    wf1 = W_f1.reshape(2, FH, FH)
    scores = _k3(
        semantic_embeddings, out_gath, den_gath, heur_pad,
        W_sem1, b_sem1.reshape(1, FH), W_sem2, b_sem2.reshape(1, FH),
        b_gat.reshape(8, 128), wstr, b_str.reshape(1, FH),
        wheur_pad, b_heur.reshape(1, 1), wf1, b_f1.reshape(1, FH),
        W_f2.reshape(1, FH), b_f2.reshape(1, 1),
        heur_logit.reshape(1, 1))
    return scores
